# Initial kernel scaffold; baseline (speedup 1.0000x reference)
import functools

import jax
import jax.numpy as jnp
from jax.experimental import pallas as pl
from jax.experimental.pallas import tpu as pltpu
from jax.experimental.pallas import tpu_sc as plsc

N_CORES = 2
N_SUB = 16
CHUNK = 128

_MESH = plsc.VectorSubcoreMesh(
    core_axis_name="c", subcore_axis_name="s", num_cores=N_CORES, num_subcores=N_SUB
)


def _deg_body(n_chunks_per_core, rows_per_tile, col_hbm, zeros_hbm, out_hbm,
              idx_v, ones_v, acc_sh, sem):
    cid = jax.lax.axis_index("c")
    sid = jax.lax.axis_index("s")

    @pl.loop(0, CHUNK)
    def _(i):
        ones_v[i, :] = jnp.full((N_SUB,), 1.0, jnp.float32)

    pltpu.sync_copy(zeros_hbm.at[pl.ds(sid * rows_per_tile, rows_per_tile)],
                    acc_sh.at[pl.ds(sid * rows_per_tile, rows_per_tile)])
    plsc.subcore_barrier()

    @pl.loop(sid, n_chunks_per_core, step=N_SUB)
    def _(k):
        base = cid * (n_chunks_per_core * CHUNK) + k * CHUNK
        pltpu.sync_copy(col_hbm.at[pl.ds(base, CHUNK)], idx_v)
        pltpu.sync_copy(ones_v, acc_sh.at[idx_v], add=True)

    plsc.subcore_barrier()
    pltpu.sync_copy(acc_sh.at[pl.ds(sid * rows_per_tile, rows_per_tile)],
                    out_hbm.at[cid, pl.ds(sid * rows_per_tile, rows_per_tile)])


def _sc_degree(col, zeros16, n, e):
    n_chunks_per_core = e // (N_CORES * CHUNK)
    rows_per_tile = n // N_SUB
    body = functools.partial(_deg_body, n_chunks_per_core, rows_per_tile)
    return pl.kernel(
        body,
        out_type=jax.ShapeDtypeStruct((N_CORES, n, N_SUB), jnp.float32),
        mesh=_MESH,
        scratch_types=[
            pltpu.VMEM((CHUNK,), jnp.int32),
            pltpu.VMEM((CHUNK, N_SUB), jnp.float32),
            pltpu.VMEM_SHARED((n, N_SUB), jnp.float32),
            pltpu.SemaphoreType.DMA,
        ],
    )(col, zeros16)


def _hop_body(n_chunks_per_core, rows_per_tile, g_hbm, row_hbm, col_hbm,
              zeros_hbm, out_hbm, idx_r, idx_c, rows_v, acc_sh, sem):
    cid = jax.lax.axis_index("c")
    sid = jax.lax.axis_index("s")

    pltpu.sync_copy(zeros_hbm.at[pl.ds(sid * rows_per_tile, rows_per_tile)],
                    acc_sh.at[pl.ds(sid * rows_per_tile, rows_per_tile)])
    plsc.subcore_barrier()

    @pl.loop(sid, n_chunks_per_core, step=N_SUB)
    def _(k):
        base = cid * (n_chunks_per_core * CHUNK) + k * CHUNK
        pltpu.sync_copy(row_hbm.at[pl.ds(base, CHUNK)], idx_r)
        pltpu.sync_copy(col_hbm.at[pl.ds(base, CHUNK)], idx_c)
        pltpu.async_copy(g_hbm.at[idx_r], rows_v, sem).wait()
        pltpu.sync_copy(rows_v, acc_sh.at[idx_c], add=True)

    plsc.subcore_barrier()
    pltpu.sync_copy(acc_sh.at[pl.ds(sid * rows_per_tile, rows_per_tile)],
                    out_hbm.at[cid, pl.ds(sid * rows_per_tile, rows_per_tile)])


def _sc_hop(g, row, col, zeros_nd, n, d, e):
    n_chunks_per_core = e // (N_CORES * CHUNK)
    rows_per_tile = n // N_SUB
    body = functools.partial(_hop_body, n_chunks_per_core, rows_per_tile)
    return pl.kernel(
        body,
        out_type=jax.ShapeDtypeStruct((N_CORES, n, d), jnp.float32),
        mesh=_MESH,
        scratch_types=[
            pltpu.VMEM((CHUNK,), jnp.int32),
            pltpu.VMEM((CHUNK,), jnp.int32),
            pltpu.VMEM((CHUNK, d), jnp.float32),
            pltpu.VMEM_SHARED((n, d), jnp.float32),
            pltpu.SemaphoreType.DMA,
        ],
    )(g, row, col, zeros_nd)


_BLK = 1000


def _mm_body(x_ref, w_ref, y_ref):
    y_ref[...] = jax.lax.dot_general(
        x_ref[...], w_ref[...], (((1,), (1,)), ((), ())),
        preferred_element_type=jnp.float32)


def _tc_matmul(x, w, n, d):
    grid = n // _BLK
    return pl.pallas_call(
        _mm_body,
        grid=(grid,),
        in_specs=[
            pl.BlockSpec((_BLK, d), lambda i: (i, 0)),
            pl.BlockSpec((d, d), lambda i: (0, 0)),
        ],
        out_specs=pl.BlockSpec((_BLK, d), lambda i: (i, 0)),
        out_shape=jax.ShapeDtypeStruct((n, d), jnp.float32),
    )(x, w)


def _scale_body(degp_ref, y_ref, g0_ref, dinv_ref, invdeg_ref):
    deg = degp_ref[0][:, 0:1] + degp_ref[1][:, 0:1] + 1.0
    dinv = jax.lax.rsqrt(deg)
    g0_ref[...] = y_ref[...] * dinv
    dinv_ref[...] = dinv
    invdeg_ref[...] = 1.0 / deg


def _tc_scale(degpart, y, n, d):
    grid = n // _BLK
    return pl.pallas_call(
        _scale_body,
        grid=(grid,),
        in_specs=[
            pl.BlockSpec((N_CORES, _BLK, N_SUB), lambda i: (0, i, 0)),
            pl.BlockSpec((_BLK, d), lambda i: (i, 0)),
        ],
        out_specs=[
            pl.BlockSpec((_BLK, d), lambda i: (i, 0)),
            pl.BlockSpec((_BLK, 1), lambda i: (i, 0)),
            pl.BlockSpec((_BLK, 1), lambda i: (i, 0)),
        ],
        out_shape=[
            jax.ShapeDtypeStruct((n, d), jnp.float32),
            jax.ShapeDtypeStruct((n, 1), jnp.float32),
            jax.ShapeDtypeStruct((n, 1), jnp.float32),
        ],
    )(degpart, y)


def _combine_body(part_ref, g_ref, scale_ref, out_ref):
    out_ref[...] = (part_ref[0] + part_ref[1] + g_ref[...]) * scale_ref[...]


def _tc_combine(part, g, scale, n, d):
    grid = n // _BLK
    return pl.pallas_call(
        _combine_body,
        grid=(grid,),
        in_specs=[
            pl.BlockSpec((N_CORES, _BLK, d), lambda i: (0, i, 0)),
            pl.BlockSpec((_BLK, d), lambda i: (i, 0)),
            pl.BlockSpec((_BLK, 1), lambda i: (i, 0)),
        ],
        out_specs=pl.BlockSpec((_BLK, d), lambda i: (i, 0)),
        out_shape=jax.ShapeDtypeStruct((n, d), jnp.float32),
    )(part, g, scale)


def _final_body(part_ref, g_ref, dinv_ref, b_ref, out_ref):
    h = (part_ref[0] + part_ref[1] + g_ref[...]) * dinv_ref[...]
    out_ref[...] = jnp.maximum(h + b_ref[...], 0.0)


def _tc_final(part, g, dinv, b2, n, d):
    grid = n // _BLK
    return pl.pallas_call(
        _final_body,
        grid=(grid,),
        in_specs=[
            pl.BlockSpec((N_CORES, _BLK, d), lambda i: (0, i, 0)),
            pl.BlockSpec((_BLK, d), lambda i: (i, 0)),
            pl.BlockSpec((_BLK, 1), lambda i: (i, 0)),
            pl.BlockSpec((1, d), lambda i: (0, 0)),
        ],
        out_specs=pl.BlockSpec((_BLK, d), lambda i: (i, 0)),
        out_shape=jax.ShapeDtypeStruct((n, d), jnp.float32),
    )(part, g, dinv, b2)


def kernel(x, edge_index, W, b):
    n, d = x.shape
    e = edge_index.shape[1]
    assert e % (N_CORES * CHUNK) == 0 and n % N_SUB == 0 and n % _BLK == 0

    row = edge_index[0]
    col = edge_index[1]
    zeros16 = jnp.zeros((n, N_SUB), jnp.float32)
    zeros_nd = jnp.zeros((n, d), jnp.float32)

    degpart = _sc_degree(col, zeros16, n, e)
    y = _tc_matmul(x, W, n, d)
    g0, dinv, invdeg = _tc_scale(degpart, y, n, d)
    part1 = _sc_hop(g0, row, col, zeros_nd, n, d, e)
    g1 = _tc_combine(part1, g0, invdeg, n, d)
    part2 = _sc_hop(g1, row, col, zeros_nd, n, d, e)
    return _tc_final(part2, g1, dinv, b.reshape(1, d), n, d)


# speedup vs baseline: 15.7201x; 15.7201x over previous
import functools

import jax
import jax.numpy as jnp
from jax.experimental import pallas as pl
from jax.experimental.pallas import tpu as pltpu
from jax.experimental.pallas import tpu_sc as plsc

N_CORES = 2
N_SUB = 16
CHUNK = 128

_MESH = plsc.VectorSubcoreMesh(
    core_axis_name="c", subcore_axis_name="s", num_cores=N_CORES, num_subcores=N_SUB
)


ROWCHUNK = 80


def _deg_body(n_chunks_per_core, n_rowchunks, col_hbm, zeros_hbm, out_hbm,
              idx_v, ones_v, acc_sh, sem):
    cid = jax.lax.axis_index("c")
    sid = jax.lax.axis_index("s")

    @pl.loop(0, CHUNK)
    def _(i):
        ones_v[i, :] = jnp.full((N_SUB,), 1.0, jnp.float32)

    @pl.loop(sid, n_rowchunks, step=N_SUB)
    def _(j):
        pltpu.sync_copy(zeros_hbm.at[pl.ds(j * ROWCHUNK, ROWCHUNK)],
                        acc_sh.at[pl.ds(j * ROWCHUNK, ROWCHUNK)])
    plsc.subcore_barrier()

    @pl.loop(sid, n_chunks_per_core, step=N_SUB)
    def _(k):
        base = cid * (n_chunks_per_core * CHUNK) + k * CHUNK
        pltpu.sync_copy(col_hbm.at[pl.ds(base, CHUNK)], idx_v)
        pltpu.sync_copy(ones_v, acc_sh.at[idx_v], add=True)

    plsc.subcore_barrier()

    @pl.loop(sid, n_rowchunks, step=N_SUB)
    def _(j):
        pltpu.sync_copy(acc_sh.at[pl.ds(j * ROWCHUNK, ROWCHUNK)],
                        out_hbm.at[cid, pl.ds(j * ROWCHUNK, ROWCHUNK)])


def _sc_degree(col, zeros16, n, e):
    n_chunks_per_core = e // (N_CORES * CHUNK)
    n_rowchunks = n // ROWCHUNK
    body = functools.partial(_deg_body, n_chunks_per_core, n_rowchunks)
    return pl.kernel(
        body,
        out_type=jax.ShapeDtypeStruct((N_CORES, n, N_SUB), jnp.float32),
        mesh=_MESH,
        scratch_types=[
            pltpu.VMEM((CHUNK,), jnp.int32),
            pltpu.VMEM((CHUNK, N_SUB), jnp.float32),
            pltpu.VMEM_SHARED((n, N_SUB), jnp.float32),
            pltpu.SemaphoreType.DMA,
        ],
    )(col, zeros16)


def _hop_body(n_chunks_per_core, n_rowchunks, g_hbm, row_hbm, col_hbm,
              zeros_hbm, out_hbm, idx_r, idx_c, rows_v, acc_sh, sem):
    cid = jax.lax.axis_index("c")
    sid = jax.lax.axis_index("s")

    @pl.loop(sid, n_rowchunks, step=N_SUB)
    def _(j):
        pltpu.sync_copy(zeros_hbm.at[pl.ds(j * ROWCHUNK, ROWCHUNK)],
                        acc_sh.at[pl.ds(j * ROWCHUNK, ROWCHUNK)])
    plsc.subcore_barrier()

    @pl.loop(sid, n_chunks_per_core, step=N_SUB)
    def _(k):
        base = cid * (n_chunks_per_core * CHUNK) + k * CHUNK
        pltpu.sync_copy(row_hbm.at[pl.ds(base, CHUNK)], idx_r)
        pltpu.sync_copy(col_hbm.at[pl.ds(base, CHUNK)], idx_c)
        pltpu.async_copy(g_hbm.at[idx_r], rows_v, sem).wait()
        pltpu.sync_copy(rows_v, acc_sh.at[idx_c], add=True)

    plsc.subcore_barrier()

    @pl.loop(sid, n_rowchunks, step=N_SUB)
    def _(j):
        pltpu.sync_copy(acc_sh.at[pl.ds(j * ROWCHUNK, ROWCHUNK)],
                        out_hbm.at[cid, pl.ds(j * ROWCHUNK, ROWCHUNK)])


def _sc_hop(g, row, col, zeros_nd, n, d, e):
    n_chunks_per_core = e // (N_CORES * CHUNK)
    n_rowchunks = n // ROWCHUNK
    body = functools.partial(_hop_body, n_chunks_per_core, n_rowchunks)
    return pl.kernel(
        body,
        out_type=jax.ShapeDtypeStruct((N_CORES, n, d), jnp.float32),
        mesh=_MESH,
        scratch_types=[
            pltpu.VMEM((CHUNK,), jnp.int32),
            pltpu.VMEM((CHUNK,), jnp.int32),
            pltpu.VMEM((CHUNK, d), jnp.float32),
            pltpu.VMEM_SHARED((n, d), jnp.float32),
            pltpu.SemaphoreType.DMA,
        ],
    )(g, row, col, zeros_nd)


_BLK = 1000


def _mm_body(x_ref, w_ref, y_ref):
    y_ref[...] = jax.lax.dot_general(
        x_ref[...], w_ref[...], (((1,), (1,)), ((), ())),
        preferred_element_type=jnp.float32)


def _tc_matmul(x, w, n, d):
    grid = n // _BLK
    return pl.pallas_call(
        _mm_body,
        grid=(grid,),
        in_specs=[
            pl.BlockSpec((_BLK, d), lambda i: (i, 0)),
            pl.BlockSpec((d, d), lambda i: (0, 0)),
        ],
        out_specs=pl.BlockSpec((_BLK, d), lambda i: (i, 0)),
        out_shape=jax.ShapeDtypeStruct((n, d), jnp.float32),
    )(x, w)


def _scale_body(degp_ref, y_ref, g0_ref, dinv_ref, invdeg_ref):
    deg = degp_ref[0][:, 0:1] + degp_ref[1][:, 0:1] + 1.0
    dinv = jax.lax.rsqrt(deg)
    g0_ref[...] = y_ref[...] * dinv
    dinv_ref[...] = dinv
    invdeg_ref[...] = 1.0 / deg


def _tc_scale(degpart, y, n, d):
    grid = n // _BLK
    return pl.pallas_call(
        _scale_body,
        grid=(grid,),
        in_specs=[
            pl.BlockSpec((N_CORES, _BLK, N_SUB), lambda i: (0, i, 0)),
            pl.BlockSpec((_BLK, d), lambda i: (i, 0)),
        ],
        out_specs=[
            pl.BlockSpec((_BLK, d), lambda i: (i, 0)),
            pl.BlockSpec((_BLK, 1), lambda i: (i, 0)),
            pl.BlockSpec((_BLK, 1), lambda i: (i, 0)),
        ],
        out_shape=[
            jax.ShapeDtypeStruct((n, d), jnp.float32),
            jax.ShapeDtypeStruct((n, 1), jnp.float32),
            jax.ShapeDtypeStruct((n, 1), jnp.float32),
        ],
    )(degpart, y)


def _combine_body(part_ref, g_ref, scale_ref, out_ref):
    out_ref[...] = (part_ref[0] + part_ref[1] + g_ref[...]) * scale_ref[...]


def _tc_combine(part, g, scale, n, d):
    grid = n // _BLK
    return pl.pallas_call(
        _combine_body,
        grid=(grid,),
        in_specs=[
            pl.BlockSpec((N_CORES, _BLK, d), lambda i: (0, i, 0)),
            pl.BlockSpec((_BLK, d), lambda i: (i, 0)),
            pl.BlockSpec((_BLK, 1), lambda i: (i, 0)),
        ],
        out_specs=pl.BlockSpec((_BLK, d), lambda i: (i, 0)),
        out_shape=jax.ShapeDtypeStruct((n, d), jnp.float32),
    )(part, g, scale)


def _final_body(part_ref, g_ref, dinv_ref, b_ref, out_ref):
    h = (part_ref[0] + part_ref[1] + g_ref[...]) * dinv_ref[...]
    out_ref[...] = jnp.maximum(h + b_ref[...], 0.0)


def _tc_final(part, g, dinv, b2, n, d):
    grid = n // _BLK
    return pl.pallas_call(
        _final_body,
        grid=(grid,),
        in_specs=[
            pl.BlockSpec((N_CORES, _BLK, d), lambda i: (0, i, 0)),
            pl.BlockSpec((_BLK, d), lambda i: (i, 0)),
            pl.BlockSpec((_BLK, 1), lambda i: (i, 0)),
            pl.BlockSpec((1, d), lambda i: (0, 0)),
        ],
        out_specs=pl.BlockSpec((_BLK, d), lambda i: (i, 0)),
        out_shape=jax.ShapeDtypeStruct((n, d), jnp.float32),
    )(part, g, dinv, b2)


def kernel(x, edge_index, W, b):
    n, d = x.shape
    e = edge_index.shape[1]
    assert e % (N_CORES * CHUNK) == 0 and n % ROWCHUNK == 0 and n % _BLK == 0

    row = edge_index[0]
    col = edge_index[1]
    zeros16 = jnp.zeros((n, N_SUB), jnp.float32)
    zeros_nd = jnp.zeros((n, d), jnp.float32)

    degpart = _sc_degree(col, zeros16, n, e)
    y = _tc_matmul(x, W, n, d)
    g0, dinv, invdeg = _tc_scale(degpart, y, n, d)
    part1 = _sc_hop(g0, row, col, zeros_nd, n, d, e)
    g1 = _tc_combine(part1, g0, invdeg, n, d)
    part2 = _sc_hop(g1, row, col, zeros_nd, n, d, e)
    return _tc_final(part2, g1, dinv, b.reshape(1, d), n, d)


# speedup vs baseline: 24.7855x; 1.5767x over previous
import functools

import jax
import jax.numpy as jnp
from jax.experimental import pallas as pl
from jax.experimental.pallas import tpu as pltpu
from jax.experimental.pallas import tpu_sc as plsc

N_CORES = 2
N_SUB = 16
CHUNK = 128

_MESH = plsc.VectorSubcoreMesh(
    core_axis_name="c", subcore_axis_name="s", num_cores=N_CORES, num_subcores=N_SUB
)


ROWCHUNK = 80


def _deg_body(n_chunks_per_core, n_rowchunks, col_hbm, zeros_hbm, out_hbm,
              idx_v, ones_v, acc_sh, sem):
    cid = jax.lax.axis_index("c")
    sid = jax.lax.axis_index("s")

    @pl.loop(0, CHUNK)
    def _(i):
        ones_v[i, :] = jnp.full((N_SUB,), 1.0, jnp.float32)

    @pl.loop(sid, n_rowchunks, step=N_SUB)
    def _(j):
        pltpu.sync_copy(zeros_hbm.at[pl.ds(j * ROWCHUNK, ROWCHUNK)],
                        acc_sh.at[pl.ds(j * ROWCHUNK, ROWCHUNK)])
    plsc.subcore_barrier()

    @pl.loop(sid, n_chunks_per_core, step=N_SUB)
    def _(k):
        base = cid * (n_chunks_per_core * CHUNK) + k * CHUNK
        pltpu.sync_copy(col_hbm.at[pl.ds(base, CHUNK)], idx_v)
        pltpu.sync_copy(ones_v, acc_sh.at[idx_v], add=True)

    plsc.subcore_barrier()

    @pl.loop(sid, n_rowchunks, step=N_SUB)
    def _(j):
        pltpu.sync_copy(acc_sh.at[pl.ds(j * ROWCHUNK, ROWCHUNK)],
                        out_hbm.at[cid, pl.ds(j * ROWCHUNK, ROWCHUNK)])


def _sc_degree(col, zeros16, n, e):
    n_chunks_per_core = e // (N_CORES * CHUNK)
    n_rowchunks = n // ROWCHUNK
    body = functools.partial(_deg_body, n_chunks_per_core, n_rowchunks)
    return pl.kernel(
        body,
        out_type=jax.ShapeDtypeStruct((N_CORES, n, N_SUB), jnp.float32),
        mesh=_MESH,
        scratch_types=[
            pltpu.VMEM((CHUNK,), jnp.int32),
            pltpu.VMEM((CHUNK, N_SUB), jnp.float32),
            pltpu.VMEM_SHARED((n, N_SUB), jnp.float32),
            pltpu.SemaphoreType.DMA,
        ],
    )(col, zeros16)


def _hop_body(n_chunks_per_core, n_rowchunks, n_iters, g_hbm, row_hbm, col_hbm,
              zeros_hbm, out_hbm, idx_r0, idx_r1, idx_c0, idx_c1, rows0, rows1,
              acc_sh, semi0, semi1, semg0, semg1, sems0, sems1):
    cid = jax.lax.axis_index("c")
    sid = jax.lax.axis_index("s")
    nc = n_chunks_per_core
    base_core = cid * (nc * CHUNK)
    slots = ((idx_r0, idx_c0, rows0, semi0, semg0, sems0),
             (idx_r1, idx_c1, rows1, semi1, semg1, sems1))

    @pl.loop(sid, n_rowchunks, step=N_SUB)
    def _(j):
        pltpu.sync_copy(zeros_hbm.at[pl.ds(j * ROWCHUNK, ROWCHUNK)],
                        acc_sh.at[pl.ds(j * ROWCHUNK, ROWCHUNK)])
    plsc.subcore_barrier()

    def issue_idx(j, irf, icf, smi):
        k = sid + j * N_SUB

        @pl.when(k < nc)
        def _():
            base = base_core + k * CHUNK
            pltpu.async_copy(row_hbm.at[pl.ds(base, CHUNK)], irf, smi)
            pltpu.async_copy(col_hbm.at[pl.ds(base, CHUNK)], icf, smi)

    issue_idx(0, idx_r0, idx_c0, semi0)
    issue_idx(1, idx_r1, idx_c1, semi1)

    @pl.loop(0, n_iters, step=2)
    def _(j0):
        for b in range(2):
            irf, icf, rwf, smi, smg, sms = slots[b]
            j = j0 + b
            k = sid + j * N_SUB

            @pl.when(k < nc)
            def _():
                base = base_core + k * CHUNK

                @pl.when(j >= 2)
                def _():
                    pltpu.make_async_copy(rwf, acc_sh.at[icf], sms).wait()

                pltpu.make_async_copy(row_hbm.at[pl.ds(base, CHUNK)],
                                      irf, smi).wait()
                pltpu.make_async_copy(col_hbm.at[pl.ds(base, CHUNK)],
                                      icf, smi).wait()
                pltpu.async_copy(g_hbm.at[irf], rwf, smg).wait()
                issue_idx(j + 2, irf, icf, smi)
                pltpu.async_copy(rwf, acc_sh.at[icf], sms, add=True)

    for b in range(2):
        irf, icf, rwf, smi, smg, sms = slots[b]

        @pl.when(sid + b * N_SUB < nc)
        def _(rwf=rwf, icf=icf, sms=sms):
            pltpu.make_async_copy(rwf, acc_sh.at[icf], sms).wait()

    plsc.subcore_barrier()

    @pl.loop(sid, n_rowchunks, step=N_SUB)
    def _(j):
        pltpu.sync_copy(acc_sh.at[pl.ds(j * ROWCHUNK, ROWCHUNK)],
                        out_hbm.at[cid, pl.ds(j * ROWCHUNK, ROWCHUNK)])


def _sc_hop(g, row, col, zeros_nd, n, d, e):
    n_chunks_per_core = e // (N_CORES * CHUNK)
    n_rowchunks = n // ROWCHUNK
    n_iters = -(-n_chunks_per_core // N_SUB)
    n_iters += n_iters % 2
    body = functools.partial(_hop_body, n_chunks_per_core, n_rowchunks, n_iters)
    return pl.kernel(
        body,
        out_type=jax.ShapeDtypeStruct((N_CORES, n, d), jnp.float32),
        mesh=_MESH,
        scratch_types=[
            pltpu.VMEM((CHUNK,), jnp.int32),
            pltpu.VMEM((CHUNK,), jnp.int32),
            pltpu.VMEM((CHUNK,), jnp.int32),
            pltpu.VMEM((CHUNK,), jnp.int32),
            pltpu.VMEM((CHUNK, d), jnp.float32),
            pltpu.VMEM((CHUNK, d), jnp.float32),
            pltpu.VMEM_SHARED((n, d), jnp.float32),
            pltpu.SemaphoreType.DMA,
            pltpu.SemaphoreType.DMA,
            pltpu.SemaphoreType.DMA,
            pltpu.SemaphoreType.DMA,
            pltpu.SemaphoreType.DMA,
            pltpu.SemaphoreType.DMA,
        ],
    )(g, row, col, zeros_nd)


_BLK = 1000


def _mm_body(x_ref, w_ref, y_ref):
    y_ref[...] = jax.lax.dot_general(
        x_ref[...], w_ref[...], (((1,), (1,)), ((), ())),
        preferred_element_type=jnp.float32)


def _tc_matmul(x, w, n, d):
    grid = n // _BLK
    return pl.pallas_call(
        _mm_body,
        grid=(grid,),
        in_specs=[
            pl.BlockSpec((_BLK, d), lambda i: (i, 0)),
            pl.BlockSpec((d, d), lambda i: (0, 0)),
        ],
        out_specs=pl.BlockSpec((_BLK, d), lambda i: (i, 0)),
        out_shape=jax.ShapeDtypeStruct((n, d), jnp.float32),
    )(x, w)


def _scale_body(degp_ref, y_ref, g0_ref, dinv_ref, invdeg_ref):
    deg = degp_ref[0][:, 0:1] + degp_ref[1][:, 0:1] + 1.0
    dinv = jax.lax.rsqrt(deg)
    g0_ref[...] = y_ref[...] * dinv
    dinv_ref[...] = dinv
    invdeg_ref[...] = 1.0 / deg


def _tc_scale(degpart, y, n, d):
    grid = n // _BLK
    return pl.pallas_call(
        _scale_body,
        grid=(grid,),
        in_specs=[
            pl.BlockSpec((N_CORES, _BLK, N_SUB), lambda i: (0, i, 0)),
            pl.BlockSpec((_BLK, d), lambda i: (i, 0)),
        ],
        out_specs=[
            pl.BlockSpec((_BLK, d), lambda i: (i, 0)),
            pl.BlockSpec((_BLK, 1), lambda i: (i, 0)),
            pl.BlockSpec((_BLK, 1), lambda i: (i, 0)),
        ],
        out_shape=[
            jax.ShapeDtypeStruct((n, d), jnp.float32),
            jax.ShapeDtypeStruct((n, 1), jnp.float32),
            jax.ShapeDtypeStruct((n, 1), jnp.float32),
        ],
    )(degpart, y)


def _combine_body(part_ref, g_ref, scale_ref, out_ref):
    out_ref[...] = (part_ref[0] + part_ref[1] + g_ref[...]) * scale_ref[...]


def _tc_combine(part, g, scale, n, d):
    grid = n // _BLK
    return pl.pallas_call(
        _combine_body,
        grid=(grid,),
        in_specs=[
            pl.BlockSpec((N_CORES, _BLK, d), lambda i: (0, i, 0)),
            pl.BlockSpec((_BLK, d), lambda i: (i, 0)),
            pl.BlockSpec((_BLK, 1), lambda i: (i, 0)),
        ],
        out_specs=pl.BlockSpec((_BLK, d), lambda i: (i, 0)),
        out_shape=jax.ShapeDtypeStruct((n, d), jnp.float32),
    )(part, g, scale)


def _final_body(part_ref, g_ref, dinv_ref, b_ref, out_ref):
    h = (part_ref[0] + part_ref[1] + g_ref[...]) * dinv_ref[...]
    out_ref[...] = jnp.maximum(h + b_ref[...], 0.0)


def _tc_final(part, g, dinv, b2, n, d):
    grid = n // _BLK
    return pl.pallas_call(
        _final_body,
        grid=(grid,),
        in_specs=[
            pl.BlockSpec((N_CORES, _BLK, d), lambda i: (0, i, 0)),
            pl.BlockSpec((_BLK, d), lambda i: (i, 0)),
            pl.BlockSpec((_BLK, 1), lambda i: (i, 0)),
            pl.BlockSpec((1, d), lambda i: (0, 0)),
        ],
        out_specs=pl.BlockSpec((_BLK, d), lambda i: (i, 0)),
        out_shape=jax.ShapeDtypeStruct((n, d), jnp.float32),
    )(part, g, dinv, b2)


def kernel(x, edge_index, W, b):
    n, d = x.shape
    e = edge_index.shape[1]
    assert e % (N_CORES * CHUNK) == 0 and n % ROWCHUNK == 0 and n % _BLK == 0

    row = edge_index[0]
    col = edge_index[1]
    zeros16 = jnp.zeros((n, N_SUB), jnp.float32)
    zeros_nd = jnp.zeros((n, d), jnp.float32)

    degpart = _sc_degree(col, zeros16, n, e)
    y = _tc_matmul(x, W, n, d)
    g0, dinv, invdeg = _tc_scale(degpart, y, n, d)
    part1 = _sc_hop(g0, row, col, zeros_nd, n, d, e)
    g1 = _tc_combine(part1, g0, invdeg, n, d)
    part2 = _sc_hop(g1, row, col, zeros_nd, n, d, e)
    return _tc_final(part2, g1, dinv, b.reshape(1, d), n, d)
